# baseline (speedup 1.0000x reference)
import functools

import jax
import jax.numpy as jnp
from jax import lax
from jax.experimental import pallas as pl
from jax.experimental.pallas import tpu as pltpu
from jax.experimental.pallas import tpu_sc as plsc

_D = 32
_G = 128
_NG = 4
_R = _G * _NG


@functools.partial(jax.jit, static_argnums=(2, 3))
def _sc_gather(xf, embeds, n_rows, n_workers):
    rows_per_worker = n_rows // n_workers
    chunks = rows_per_worker // _R
    obs = n_rows // _G
    obs_per_worker = rows_per_worker // _G

    mesh = plsc.VectorSubcoreMesh(core_axis_name="c", subcore_axis_name="s")

    @functools.partial(
        pl.kernel,
        out_type=jax.ShapeDtypeStruct((_D // 8, obs, 8, _G), jnp.float32),
        mesh=mesh,
        scratch_types=[
            pltpu.VMEM((2, _R), jnp.int32),
            pltpu.VMEM((2, _R, _D), jnp.float32),
            pltpu.VMEM((2, _NG, _D, _G), jnp.float32),
            pltpu.SemaphoreType.DMA,
            pltpu.SemaphoreType.DMA,
            pltpu.SemaphoreType.DMA,
            pltpu.SemaphoreType.DMA,
            pltpu.SemaphoreType.DMA,
            pltpu.SemaphoreType.DMA,
        ],
        compiler_params=pltpu.CompilerParams(
            use_tc_tiling_on_sc=False, needs_layout_passes=False
        ),
    )
    def body(x_hbm, tab_hbm, out_hbm, idx_v, rows_v, trows_v,
             i0, i1, g0, g1, o0, o1):
        wid = lax.axis_index("s") * mesh.num_cores + lax.axis_index("c")
        row_base = wid * rows_per_worker
        ob_base = wid * obs_per_worker
        isems = (i0, i1)
        gsems = (g0, g1)
        osems = (o0, o1)

        def icopy(c, b):
            row0 = row_base + c * _R
            return pltpu.make_async_copy(
                x_hbm.at[pl.ds(row0, _R)], idx_v.at[b], isems[b]
            )

        def gcopy(b):
            return pltpu.make_async_copy(
                tab_hbm.at[idx_v.at[b]], rows_v.at[b], gsems[b]
            )

        def ocopy(c, b, g, db):
            ob = ob_base + c * _NG + g
            return pltpu.make_async_copy(
                trows_v.at[b, g, pl.ds(8 * db, 8)],
                out_hbm.at[db, ob],
                osems[b],
            )

        def transpose_and_emit(c, b, wait_out):
            if wait_out:
                for g in range(_NG):
                    for db in range(_D // 8):
                        ocopy(c, b, g, db).wait()
            for g in range(_NG):
                def col(dd, carry):
                    for jb in range(_G // 16):
                        rows16 = (g * _G + jb * 16
                                  + lax.iota(jnp.int32, 16))
                        cols16 = jnp.full((16,), dd, jnp.int32)
                        v = plsc.load_gather(rows_v.at[b], [rows16, cols16])
                        trows_v[b, g, dd, pl.ds(jb * 16, 16)] = v
                    return carry
                lax.fori_loop(0, _D, col, 0, unroll=2)
            for g in range(_NG):
                for db in range(_D // 8):
                    ocopy(c, b, g, db).start()

        icopy(0, 0).start()
        icopy(1, 1).start()
        icopy(0, 0).wait()
        gcopy(0).start()

        def step(c, b, wait_out, last):
            gcopy(b).wait()
            if not last:
                icopy(c + 1, 1 - b).wait()
                gcopy(1 - b).start()
            transpose_and_emit(c, b, wait_out)
            nxt = jnp.minimum(c + 2, chunks - 1)
            icopy(nxt, b).start()

        step(0, 0, False, False)
        step(1, 1, False, False)

        def loop(i, carry):
            step(2 * i, 0, True, False)
            step(2 * i + 1, 1, True, False)
            return carry

        lax.fori_loop(1, chunks // 2 - 1, loop, 0)
        step(chunks - 2, 0, True, False)
        step(chunks - 1, 1, True, True)

        icopy(chunks - 1, 0).wait()
        icopy(chunks - 1, 1).wait()
        for b in (0, 1):
            for g in range(_NG):
                for db in range(_D // 8):
                    ocopy(chunks - 2 + b, b, g, db).wait()

    return body(xf, embeds)


def kernel(x, embeds):
    n = x.size
    xf = x.reshape(-1).astype(jnp.int32)
    op = _sc_gather(xf, embeds, n, 32)
    return op.transpose(1, 3, 0, 2).reshape(n, _D)
